# Initial kernel scaffold; baseline (speedup 1.0000x reference)
import functools

import jax
import jax.numpy as jnp
from jax import lax
from jax.experimental import pallas as pl
from jax.experimental.pallas import tpu as pltpu
from jax.experimental.pallas import tpu_sc as plsc

NC = 2
NS = 16
NW = NC * NS
L = 16
S = 8


def _wid():
    return lax.axis_index("s") * NC + lax.axis_index("c")


def _reduce_by_8(rows_v, out_v, n_nodes, d, scale):
    nblk = d // L

    def body(n, carry):
        for db in range(nblk):
            sl = pl.ds(db * L, L)
            acc = rows_v[n * S + 0, sl]
            for k in range(1, S):
                acc = acc + rows_v[n * S + k, sl]
            out_v[n, sl] = acc * scale
        return carry

    lax.fori_loop(0, n_nodes, body, 0)


def _sage_gather_l1(n1p, b_per_w, in_dim, n0_per_w, feats, nidx, sidx1,
                    sidx0):
    n_chunks = b_per_w // L
    half = b_per_w // 2

    mesh = plsc.VectorSubcoreMesh(core_axis_name="c", subcore_axis_name="s")

    @functools.partial(
        pl.kernel,
        out_type=(
            jax.ShapeDtypeStruct((n1p, in_dim), jnp.float32),
            jax.ShapeDtypeStruct((n1p, in_dim), jnp.float32),
            jax.ShapeDtypeStruct((NW * n0_per_w, in_dim), jnp.float32),
        ),
        mesh=mesh,
        scratch_types=[
            pltpu.VMEM((L * S,), jnp.int32),
            pltpu.VMEM((L * S,), jnp.int32),
            pltpu.VMEM((L * S, in_dim), jnp.float32),
            pltpu.VMEM((L * S, in_dim), jnp.float32),
            pltpu.VMEM((L, in_dim), jnp.float32),
            pltpu.VMEM((L, in_dim), jnp.float32),
            pltpu.VMEM((half,), jnp.int32),
            pltpu.VMEM((half, in_dim), jnp.float32),
            pltpu.VMEM((n0_per_w,), jnp.int32),
            pltpu.VMEM((n0_per_w, in_dim), jnp.float32),
            pltpu.SemaphoreType.DMA,
            pltpu.SemaphoreType.DMA,
            pltpu.SemaphoreType.DMA,
            pltpu.SemaphoreType.DMA,
        ],
    )
    def k(feats_hbm, nidx_hbm, sidx1_hbm, sidx0_hbm,
          sum1_hbm, self1_hbm, self0_hbm,
          idx_a, idx_b, rows_a, rows_b, out_a, out_b,
          sidx_v, srow_v, s0idx_v, s0row_v,
          sem_a, sem_b, sem_oa, sem_ob):
        w = _wid()
        nbase = w * (b_per_w * S)
        obase = w * b_per_w

        idxs = (idx_a, idx_b)
        rows = (rows_a, rows_b)
        outs = (out_a, out_b)
        sems = (sem_a, sem_b)
        osems = (sem_oa, sem_ob)

        pltpu.sync_copy(nidx_hbm.at[pl.ds(nbase, L * S)], idx_a)
        gathers = [None, None]
        gathers[0] = pltpu.make_async_copy(feats_hbm.at[idx_a], rows_a,
                                           sem_a)
        gathers[0].start()
        owaits = [None, None]
        for c in range(n_chunks):
            cur = c % 2
            nxt = 1 - cur
            if c + 1 < n_chunks:
                pltpu.sync_copy(
                    nidx_hbm.at[pl.ds(nbase + (c + 1) * L * S, L * S)],
                    idxs[nxt])
                gathers[nxt] = pltpu.make_async_copy(
                    feats_hbm.at[idxs[nxt]], rows[nxt], sems[nxt])
                gathers[nxt].start()
            gathers[cur].wait()
            if owaits[cur] is not None:
                owaits[cur].wait()
            _reduce_by_8(rows[cur], outs[cur], L, in_dim, 1.0 / S)
            owaits[cur] = pltpu.make_async_copy(
                outs[cur], sum1_hbm.at[pl.ds(obase + c * L, L), :],
                osems[cur])
            owaits[cur].start()
        for ow in owaits:
            if ow is not None:
                ow.wait()

        for hidx in range(2):
            pltpu.sync_copy(
                sidx1_hbm.at[pl.ds(obase + hidx * half, half)], sidx_v)
            pltpu.make_async_copy(feats_hbm.at[sidx_v], srow_v,
                                  sem_a).start()
            pltpu.make_async_copy(feats_hbm.at[sidx_v], srow_v,
                                  sem_a).wait()
            pltpu.sync_copy(
                srow_v, self1_hbm.at[pl.ds(obase + hidx * half, half), :])

        sbase = w * n0_per_w
        pltpu.sync_copy(sidx0_hbm.at[pl.ds(sbase, n0_per_w)], s0idx_v)
        pltpu.make_async_copy(feats_hbm.at[s0idx_v], s0row_v, sem_b).start()
        pltpu.make_async_copy(feats_hbm.at[s0idx_v], s0row_v, sem_b).wait()
        pltpu.sync_copy(s0row_v, self0_hbm.at[pl.ds(sbase, n0_per_w), :])

    return k(feats, nidx, sidx1, sidx0)


def _sage_gather_l0(b, hid_dim, h, nidx):
    n0 = b // NW
    n_chunks = n0 // L

    mesh = plsc.VectorSubcoreMesh(core_axis_name="c", subcore_axis_name="s")

    @functools.partial(
        pl.kernel,
        out_type=jax.ShapeDtypeStruct((b, hid_dim), jnp.float32),
        mesh=mesh,
        scratch_types=[
            pltpu.VMEM((L * S,), jnp.int32),
            pltpu.VMEM((L * S,), jnp.int32),
            pltpu.VMEM((L * S, hid_dim), jnp.float32),
            pltpu.VMEM((L * S, hid_dim), jnp.float32),
            pltpu.VMEM((L, hid_dim), jnp.float32),
            pltpu.VMEM((L, hid_dim), jnp.float32),
            pltpu.SemaphoreType.DMA,
            pltpu.SemaphoreType.DMA,
            pltpu.SemaphoreType.DMA,
            pltpu.SemaphoreType.DMA,
        ],
    )
    def k(h_hbm, nidx_hbm, sum0_hbm,
          idx_a, idx_b, rows_a, rows_b, out_a, out_b,
          sem_a, sem_b, sem_oa, sem_ob):
        w = _wid()
        nbase = w * (n0 * S)
        obase = w * n0

        idxs = (idx_a, idx_b)
        rows = (rows_a, rows_b)
        outs = (out_a, out_b)
        sems = (sem_a, sem_b)
        osems = (sem_oa, sem_ob)

        pltpu.sync_copy(nidx_hbm.at[pl.ds(nbase, L * S)], idx_a)
        gathers = [None, None]
        gathers[0] = pltpu.make_async_copy(h_hbm.at[idx_a], rows_a, sem_a)
        gathers[0].start()
        owaits = [None, None]
        for c in range(n_chunks):
            cur = c % 2
            nxt = 1 - cur
            if c + 1 < n_chunks:
                pltpu.sync_copy(
                    nidx_hbm.at[pl.ds(nbase + (c + 1) * L * S, L * S)],
                    idxs[nxt])
                gathers[nxt] = pltpu.make_async_copy(
                    h_hbm.at[idxs[nxt]], rows[nxt], sems[nxt])
                gathers[nxt].start()
            gathers[cur].wait()
            if owaits[cur] is not None:
                owaits[cur].wait()
            _reduce_by_8(rows[cur], outs[cur], L, hid_dim, 1.0 / S)
            owaits[cur] = pltpu.make_async_copy(
                outs[cur], sum0_hbm.at[pl.ds(obase + c * L, L), :],
                osems[cur])
            owaits[cur].start()
        for ow in owaits:
            if ow is not None:
                ow.wait()

    return k(h, nidx)


def _agg_matmul_kernel(x_ref, m_ref, ws_ref, wn_ref, o_ref):
    o_ref[...] = jnp.maximum(
        jnp.dot(x_ref[...], ws_ref[...], preferred_element_type=jnp.float32)
        + jnp.dot(m_ref[...], wn_ref[...],
                  preferred_element_type=jnp.float32),
        0.0)


def _tc_layer1(x, m, ws, wn, bm=512):
    n1p = x.shape[0]
    hid = ws.shape[1]
    return pl.pallas_call(
        _agg_matmul_kernel,
        grid=(n1p // bm,),
        in_specs=[
            pl.BlockSpec((bm, x.shape[1]), lambda i: (i, 0)),
            pl.BlockSpec((bm, m.shape[1]), lambda i: (i, 0)),
            pl.BlockSpec(ws.shape, lambda i: (0, 0)),
            pl.BlockSpec(wn.shape, lambda i: (0, 0)),
        ],
        out_specs=pl.BlockSpec((bm, hid), lambda i: (i, 0)),
        out_shape=jax.ShapeDtypeStruct((n1p, hid), jnp.float32),
    )(x, m, ws, wn)


def _final_kernel(x_ref, m_ref, ws_ref, wn_ref, wfc_ref, b_ref, o_ref):
    h0 = jnp.maximum(
        jnp.dot(x_ref[...], ws_ref[...], preferred_element_type=jnp.float32)
        + jnp.dot(m_ref[...], wn_ref[...],
                  preferred_element_type=jnp.float32),
        0.0)
    o_ref[...] = (
        jnp.dot(h0, wfc_ref[...], preferred_element_type=jnp.float32)
        + b_ref[...])


def _tc_layer0(x, m, ws, wn, wfc, bfc):
    b = x.shape[0]
    out_dim = wfc.shape[1]
    return pl.pallas_call(
        _final_kernel,
        out_shape=jax.ShapeDtypeStruct((b, out_dim), jnp.float32),
    )(x, m, ws, wn, wfc, bfc.reshape(1, -1))


@jax.jit
def kernel(feats, nodes0, neig0, nodes1, neig1,
           W_self0, W_neigh0, W_self1, W_neigh1, W_fc, b_fc):
    n1 = nodes1.shape[0]
    in_dim = feats.shape[1]
    b = nodes0.shape[0]
    n1p = ((n1 + NW * L - 1) // (NW * L)) * (NW * L)
    pad = n1p - n1
    b_per_w = n1p // NW

    nidx1 = jnp.pad(neig1.astype(jnp.int32), ((0, pad), (0, 0))).reshape(-1)
    sidx1 = jnp.pad(nodes1.astype(jnp.int32), (0, pad))
    sidx0 = nodes0.astype(jnp.int32)
    nidx0 = neig0.astype(jnp.int32).reshape(-1)

    sum1, self1, self0 = _sage_gather_l1(
        n1p, b_per_w, in_dim, b // NW, feats, nidx1, sidx1, sidx0)
    h = _tc_layer1(self1, sum1, W_self1, W_neigh1)
    sum0 = _sage_gather_l0(b, h.shape[1], h, nidx0)
    return _tc_layer0(self0, sum0, W_self0, W_neigh0, W_fc, b_fc)


# speedup vs baseline: 1.3936x; 1.3936x over previous
import functools

import jax
import jax.numpy as jnp
from jax import lax
from jax.experimental import pallas as pl
from jax.experimental.pallas import tpu as pltpu
from jax.experimental.pallas import tpu_sc as plsc

NC = 2
NS = 16
NW = NC * NS
L = 16
S = 8


def _wid():
    return lax.axis_index("s") * NC + lax.axis_index("c")


def _reduce_by_8(rows_v, out_v, n_nodes, d, scale):
    nblk = d // L

    def body(n, carry):
        for db in range(nblk):
            sl = pl.ds(db * L, L)
            acc = rows_v[n * S + 0, sl]
            for k in range(1, S):
                acc = acc + rows_v[n * S + k, sl]
            out_v[n, sl] = acc * scale
        return carry

    lax.fori_loop(0, n_nodes, body, 0)


def _sage_gather_l1(n1p, b_per_w, in_dim, n0_per_w, feats, nidx, sidx1,
                    sidx0):
    n_chunks = b_per_w // L
    half = b_per_w // 2

    mesh = plsc.VectorSubcoreMesh(core_axis_name="c", subcore_axis_name="s")

    @functools.partial(
        pl.kernel,
        out_type=(
            jax.ShapeDtypeStruct((n1p, in_dim), jnp.float32),
            jax.ShapeDtypeStruct((n1p, in_dim), jnp.float32),
            jax.ShapeDtypeStruct((NW * n0_per_w, in_dim), jnp.float32),
        ),
        mesh=mesh,
        scratch_types=[
            pltpu.VMEM((L * S,), jnp.int32),
            pltpu.VMEM((L * S,), jnp.int32),
            pltpu.VMEM((L * S, in_dim), jnp.float32),
            pltpu.VMEM((L * S, in_dim), jnp.float32),
            pltpu.VMEM((L, in_dim), jnp.float32),
            pltpu.VMEM((L, in_dim), jnp.float32),
            pltpu.VMEM((half,), jnp.int32),
            pltpu.VMEM((half, in_dim), jnp.float32),
            pltpu.VMEM((n0_per_w,), jnp.int32),
            pltpu.VMEM((n0_per_w, in_dim), jnp.float32),
            pltpu.SemaphoreType.DMA,
            pltpu.SemaphoreType.DMA,
            pltpu.SemaphoreType.DMA,
            pltpu.SemaphoreType.DMA,
        ],
    )
    def k(feats_hbm, nidx_hbm, sidx1_hbm, sidx0_hbm,
          sum1_hbm, self1_hbm, self0_hbm,
          idx_a, idx_b, rows_a, rows_b, out_a, out_b,
          sidx_v, srow_v, s0idx_v, s0row_v,
          sem_a, sem_b, sem_oa, sem_ob):
        w = _wid()
        nbase = w * (b_per_w * S)
        obase = w * b_per_w

        idxs = (idx_a, idx_b)
        rows = (rows_a, rows_b)
        outs = (out_a, out_b)
        sems = (sem_a, sem_b)
        osems = (sem_oa, sem_ob)

        pltpu.sync_copy(nidx_hbm.at[pl.ds(nbase, L * S)], idx_a)
        gathers = [None, None]
        gathers[0] = pltpu.make_async_copy(feats_hbm.at[idx_a], rows_a,
                                           sem_a)
        gathers[0].start()
        owaits = [None, None]
        for c in range(n_chunks):
            cur = c % 2
            nxt = 1 - cur
            if c + 1 < n_chunks:
                pltpu.sync_copy(
                    nidx_hbm.at[pl.ds(nbase + (c + 1) * L * S, L * S)],
                    idxs[nxt])
                gathers[nxt] = pltpu.make_async_copy(
                    feats_hbm.at[idxs[nxt]], rows[nxt], sems[nxt])
                gathers[nxt].start()
            gathers[cur].wait()
            if owaits[cur] is not None:
                owaits[cur].wait()
            _reduce_by_8(rows[cur], outs[cur], L, in_dim, 1.0 / S)
            owaits[cur] = pltpu.make_async_copy(
                outs[cur], sum1_hbm.at[pl.ds(obase + c * L, L), :],
                osems[cur])
            owaits[cur].start()
        for ow in owaits:
            if ow is not None:
                ow.wait()

        for hidx in range(2):
            pltpu.sync_copy(
                sidx1_hbm.at[pl.ds(obase + hidx * half, half)], sidx_v)
            sgather = pltpu.make_async_copy(feats_hbm.at[sidx_v], srow_v,
                                            sem_a)
            sgather.start()
            sgather.wait()
            pltpu.sync_copy(
                srow_v, self1_hbm.at[pl.ds(obase + hidx * half, half), :])

        sbase = w * n0_per_w
        pltpu.sync_copy(sidx0_hbm.at[pl.ds(sbase, n0_per_w)], s0idx_v)
        s0gather = pltpu.make_async_copy(feats_hbm.at[s0idx_v], s0row_v,
                                         sem_b)
        s0gather.start()
        s0gather.wait()
        pltpu.sync_copy(s0row_v, self0_hbm.at[pl.ds(sbase, n0_per_w), :])

    return k(feats, nidx, sidx1, sidx0)


def _sage_gather_l0(b, hid_dim, h, nidx):
    n0 = b // NW
    n_chunks = n0 // L

    mesh = plsc.VectorSubcoreMesh(core_axis_name="c", subcore_axis_name="s")

    @functools.partial(
        pl.kernel,
        out_type=jax.ShapeDtypeStruct((b, hid_dim), jnp.float32),
        mesh=mesh,
        scratch_types=[
            pltpu.VMEM((L * S,), jnp.int32),
            pltpu.VMEM((L * S,), jnp.int32),
            pltpu.VMEM((L * S, hid_dim), jnp.float32),
            pltpu.VMEM((L * S, hid_dim), jnp.float32),
            pltpu.VMEM((L, hid_dim), jnp.float32),
            pltpu.VMEM((L, hid_dim), jnp.float32),
            pltpu.SemaphoreType.DMA,
            pltpu.SemaphoreType.DMA,
            pltpu.SemaphoreType.DMA,
            pltpu.SemaphoreType.DMA,
        ],
    )
    def k(h_hbm, nidx_hbm, sum0_hbm,
          idx_a, idx_b, rows_a, rows_b, out_a, out_b,
          sem_a, sem_b, sem_oa, sem_ob):
        w = _wid()
        nbase = w * (n0 * S)
        obase = w * n0

        idxs = (idx_a, idx_b)
        rows = (rows_a, rows_b)
        outs = (out_a, out_b)
        sems = (sem_a, sem_b)
        osems = (sem_oa, sem_ob)

        pltpu.sync_copy(nidx_hbm.at[pl.ds(nbase, L * S)], idx_a)
        gathers = [None, None]
        gathers[0] = pltpu.make_async_copy(h_hbm.at[idx_a], rows_a, sem_a)
        gathers[0].start()
        owaits = [None, None]
        for c in range(n_chunks):
            cur = c % 2
            nxt = 1 - cur
            if c + 1 < n_chunks:
                pltpu.sync_copy(
                    nidx_hbm.at[pl.ds(nbase + (c + 1) * L * S, L * S)],
                    idxs[nxt])
                gathers[nxt] = pltpu.make_async_copy(
                    h_hbm.at[idxs[nxt]], rows[nxt], sems[nxt])
                gathers[nxt].start()
            gathers[cur].wait()
            if owaits[cur] is not None:
                owaits[cur].wait()
            _reduce_by_8(rows[cur], outs[cur], L, hid_dim, 1.0 / S)
            owaits[cur] = pltpu.make_async_copy(
                outs[cur], sum0_hbm.at[pl.ds(obase + c * L, L), :],
                osems[cur])
            owaits[cur].start()
        for ow in owaits:
            if ow is not None:
                ow.wait()

    return k(h, nidx)


def _agg_matmul_kernel(x_ref, m_ref, ws_ref, wn_ref, o_ref):
    o_ref[...] = jnp.maximum(
        jnp.dot(x_ref[...], ws_ref[...], preferred_element_type=jnp.float32)
        + jnp.dot(m_ref[...], wn_ref[...],
                  preferred_element_type=jnp.float32),
        0.0)


def _tc_layer1(x, m, ws, wn, bm=512):
    n1p = x.shape[0]
    hid = ws.shape[1]
    return pl.pallas_call(
        _agg_matmul_kernel,
        grid=(n1p // bm,),
        in_specs=[
            pl.BlockSpec((bm, x.shape[1]), lambda i: (i, 0)),
            pl.BlockSpec((bm, m.shape[1]), lambda i: (i, 0)),
            pl.BlockSpec(ws.shape, lambda i: (0, 0)),
            pl.BlockSpec(wn.shape, lambda i: (0, 0)),
        ],
        out_specs=pl.BlockSpec((bm, hid), lambda i: (i, 0)),
        out_shape=jax.ShapeDtypeStruct((n1p, hid), jnp.float32),
    )(x, m, ws, wn)


def _final_kernel(x_ref, m_ref, ws_ref, wn_ref, wfc_ref, b_ref, o_ref):
    h0 = jnp.maximum(
        jnp.dot(x_ref[...], ws_ref[...], preferred_element_type=jnp.float32)
        + jnp.dot(m_ref[...], wn_ref[...],
                  preferred_element_type=jnp.float32),
        0.0)
    o_ref[...] = (
        jnp.dot(h0, wfc_ref[...], preferred_element_type=jnp.float32)
        + b_ref[...])


def _tc_layer0(x, m, ws, wn, wfc, bfc):
    b = x.shape[0]
    out_dim = wfc.shape[1]
    return pl.pallas_call(
        _final_kernel,
        out_shape=jax.ShapeDtypeStruct((b, out_dim), jnp.float32),
    )(x, m, ws, wn, wfc, bfc.reshape(1, -1))


@jax.jit
def kernel(feats, nodes0, neig0, nodes1, neig1,
           W_self0, W_neigh0, W_self1, W_neigh1, W_fc, b_fc):
    n1 = nodes1.shape[0]
    in_dim = feats.shape[1]
    b = nodes0.shape[0]
    n1p = ((n1 + NW * L - 1) // (NW * L)) * (NW * L)
    pad = n1p - n1
    b_per_w = n1p // NW

    nidx1 = jnp.pad(neig1.astype(jnp.int32), ((0, pad), (0, 0))).reshape(-1)
    sidx1 = jnp.pad(nodes1.astype(jnp.int32), (0, pad))
    sidx0 = nodes0.astype(jnp.int32)
    nidx0 = neig0.astype(jnp.int32).reshape(-1)

    sum1, self1, self0 = _sage_gather_l1(
        n1p, b_per_w, in_dim, b // NW, feats, nidx1, sidx1, sidx0)
    h = _tc_layer1(self1, sum1, W_self1, W_neigh1)
    sum0 = _sage_gather_l0(b, h.shape[1], h, nidx0)
    return _tc_layer0(self0, sum0, W_self0, W_neigh0, W_fc, b_fc)


# speedup vs baseline: 1.4847x; 1.0653x over previous
import functools

import jax
import jax.numpy as jnp
from jax import lax
from jax.experimental import pallas as pl
from jax.experimental.pallas import tpu as pltpu
from jax.experimental.pallas import tpu_sc as plsc

NC = 2
NS = 16
NW = NC * NS
L = 16
S = 8
NBUF = 3


def _wid():
    return lax.axis_index("s") * NC + lax.axis_index("c")


def _reduce_by_8(rows_v, out_v, n_nodes, d, scale):
    nblk = d // L

    def body(n, carry):
        base = n * S
        for db in range(nblk):
            sl = pl.ds(db * L, L)
            s0 = rows_v[base + 0, sl] + rows_v[base + 1, sl]
            s1 = rows_v[base + 2, sl] + rows_v[base + 3, sl]
            s2 = rows_v[base + 4, sl] + rows_v[base + 5, sl]
            s3 = rows_v[base + 6, sl] + rows_v[base + 7, sl]
            out_v[n, sl] = ((s0 + s1) + (s2 + s3)) * scale
        return carry

    lax.fori_loop(0, n_nodes, body, 0)


def _sage_gather_l1(n1p, b_per_w, in_dim, n0_per_w, feats, nidx, sidx1,
                    sidx0):
    n_chunks = b_per_w // L
    half = b_per_w // 2

    mesh = plsc.VectorSubcoreMesh(core_axis_name="c", subcore_axis_name="s")

    @functools.partial(
        pl.kernel,
        out_type=(
            jax.ShapeDtypeStruct((n1p, in_dim), jnp.float32),
            jax.ShapeDtypeStruct((n1p, in_dim), jnp.float32),
            jax.ShapeDtypeStruct((NW * n0_per_w, in_dim), jnp.float32),
        ),
        mesh=mesh,
        scratch_types=[
            pltpu.VMEM((n_chunks * L * S,), jnp.int32),
            pltpu.VMEM((2 * half,), jnp.int32),
            pltpu.VMEM((n0_per_w,), jnp.int32),
            [pltpu.VMEM((L * S, in_dim), jnp.float32)] * NBUF,
            [pltpu.VMEM((L, in_dim), jnp.float32)] * 2,
            [pltpu.VMEM((half, in_dim), jnp.float32)] * 2,
            pltpu.VMEM((n0_per_w, in_dim), jnp.float32),
            [pltpu.SemaphoreType.DMA] * NBUF,
            [pltpu.SemaphoreType.DMA] * 2,
            [pltpu.SemaphoreType.DMA] * 3,
        ],
    )
    def k(feats_hbm, nidx_hbm, sidx1_hbm, sidx0_hbm,
          sum1_hbm, self1_hbm, self0_hbm,
          nidx_v, sidx_v, s0idx_v, rows, outs, srows, s0row_v,
          gsems, osems, ssems):
        w = _wid()
        obase = w * b_per_w

        pltpu.sync_copy(
            nidx_hbm.at[pl.ds(w * n_chunks * L * S, n_chunks * L * S)],
            nidx_v)
        pltpu.sync_copy(sidx1_hbm.at[pl.ds(w * 2 * half, 2 * half)], sidx_v)
        pltpu.sync_copy(sidx0_hbm.at[pl.ds(w * n0_per_w, n0_per_w)],
                        s0idx_v)

        sg = [pltpu.make_async_copy(
            feats_hbm.at[sidx_v.at[pl.ds(i * half, half)]], srows[i],
            ssems[i]) for i in range(2)]
        sg[0].start()
        sg[1].start()
        s0g = pltpu.make_async_copy(feats_hbm.at[s0idx_v], s0row_v,
                                    ssems[2])
        s0g.start()

        gathers = [None] * NBUF
        for c in range(min(NBUF, n_chunks)):
            gathers[c] = pltpu.make_async_copy(
                feats_hbm.at[nidx_v.at[pl.ds(c * L * S, L * S)]], rows[c],
                gsems[c])
            gathers[c].start()
        owaits = [None, None]
        for c in range(n_chunks):
            cur = c % NBUF
            gathers[cur].wait()
            nx = c + NBUF
            ocur = c % 2
            if owaits[ocur] is not None:
                owaits[ocur].wait()
            _reduce_by_8(rows[cur], outs[ocur], L, in_dim, 1.0 / S)
            if nx < n_chunks:
                gathers[cur] = pltpu.make_async_copy(
                    feats_hbm.at[nidx_v.at[pl.ds(nx * L * S, L * S)]],
                    rows[cur], gsems[cur])
                gathers[cur].start()
            owaits[ocur] = pltpu.make_async_copy(
                outs[ocur], sum1_hbm.at[pl.ds(obase + c * L, L), :],
                osems[ocur])
            owaits[ocur].start()
        for ow in owaits:
            if ow is not None:
                ow.wait()

        for i in range(2):
            sg[i].wait()
            pltpu.sync_copy(
                srows[i], self1_hbm.at[pl.ds(obase + i * half, half), :])
        sbase = w * n0_per_w
        s0g.wait()
        pltpu.sync_copy(s0row_v, self0_hbm.at[pl.ds(sbase, n0_per_w), :])

    return k(feats, nidx, sidx1, sidx0)


def _sage_gather_l0(b, hid_dim, h, nidx):
    n0 = b // NW
    n_chunks = n0 // L

    mesh = plsc.VectorSubcoreMesh(core_axis_name="c", subcore_axis_name="s")

    @functools.partial(
        pl.kernel,
        out_type=jax.ShapeDtypeStruct((b, hid_dim), jnp.float32),
        mesh=mesh,
        scratch_types=[
            pltpu.VMEM((n_chunks * L * S,), jnp.int32),
            [pltpu.VMEM((L * S, hid_dim), jnp.float32)] * 2,
            [pltpu.VMEM((L, hid_dim), jnp.float32)] * 2,
            [pltpu.SemaphoreType.DMA] * 2,
            [pltpu.SemaphoreType.DMA] * 2,
        ],
    )
    def k(h_hbm, nidx_hbm, sum0_hbm, nidx_v, rows, outs, gsems, osems):
        w = _wid()
        obase = w * n0

        pltpu.sync_copy(
            nidx_hbm.at[pl.ds(w * n_chunks * L * S, n_chunks * L * S)],
            nidx_v)
        gathers = [None] * n_chunks
        for c in range(n_chunks):
            gathers[c] = pltpu.make_async_copy(
                h_hbm.at[nidx_v.at[pl.ds(c * L * S, L * S)]], rows[c],
                gsems[c])
            gathers[c].start()
        owaits = [None] * n_chunks
        for c in range(n_chunks):
            gathers[c].wait()
            _reduce_by_8(rows[c], outs[c], L, hid_dim, 1.0 / S)
            owaits[c] = pltpu.make_async_copy(
                outs[c], sum0_hbm.at[pl.ds(obase + c * L, L), :],
                osems[c])
            owaits[c].start()
        for ow in owaits:
            ow.wait()

    return k(h, nidx)


def _agg_matmul_kernel(x_ref, m_ref, ws_ref, wn_ref, o_ref):
    o_ref[...] = jnp.maximum(
        jnp.dot(x_ref[...], ws_ref[...], preferred_element_type=jnp.float32)
        + jnp.dot(m_ref[...], wn_ref[...],
                  preferred_element_type=jnp.float32),
        0.0)


def _tc_layer1(x, m, ws, wn, bm=512):
    n1p = x.shape[0]
    hid = ws.shape[1]
    return pl.pallas_call(
        _agg_matmul_kernel,
        grid=(n1p // bm,),
        in_specs=[
            pl.BlockSpec((bm, x.shape[1]), lambda i: (i, 0)),
            pl.BlockSpec((bm, m.shape[1]), lambda i: (i, 0)),
            pl.BlockSpec(ws.shape, lambda i: (0, 0)),
            pl.BlockSpec(wn.shape, lambda i: (0, 0)),
        ],
        out_specs=pl.BlockSpec((bm, hid), lambda i: (i, 0)),
        out_shape=jax.ShapeDtypeStruct((n1p, hid), jnp.float32),
    )(x, m, ws, wn)


def _final_kernel(x_ref, m_ref, ws_ref, wn_ref, wfc_ref, b_ref, o_ref):
    h0 = jnp.maximum(
        jnp.dot(x_ref[...], ws_ref[...], preferred_element_type=jnp.float32)
        + jnp.dot(m_ref[...], wn_ref[...],
                  preferred_element_type=jnp.float32),
        0.0)
    o_ref[...] = (
        jnp.dot(h0, wfc_ref[...], preferred_element_type=jnp.float32)
        + b_ref[...])


def _tc_layer0(x, m, ws, wn, wfc, bfc):
    b = x.shape[0]
    out_dim = wfc.shape[1]
    return pl.pallas_call(
        _final_kernel,
        out_shape=jax.ShapeDtypeStruct((b, out_dim), jnp.float32),
    )(x, m, ws, wn, wfc, bfc.reshape(1, -1))


@jax.jit
def kernel(feats, nodes0, neig0, nodes1, neig1,
           W_self0, W_neigh0, W_self1, W_neigh1, W_fc, b_fc):
    n1 = nodes1.shape[0]
    in_dim = feats.shape[1]
    b = nodes0.shape[0]
    n1p = ((n1 + NW * L - 1) // (NW * L)) * (NW * L)
    pad = n1p - n1
    b_per_w = n1p // NW

    nidx1 = jnp.pad(neig1.astype(jnp.int32), ((0, pad), (0, 0))).reshape(-1)
    sidx1 = jnp.pad(nodes1.astype(jnp.int32), (0, pad))
    sidx0 = nodes0.astype(jnp.int32)
    nidx0 = neig0.astype(jnp.int32).reshape(-1)

    sum1, self1, self0 = _sage_gather_l1(
        n1p, b_per_w, in_dim, b // NW, feats, nidx1, sidx1, sidx0)
    h = _tc_layer1(self1, sum1, W_self1, W_neigh1)
    sum0 = _sage_gather_l0(b, h.shape[1], h, nidx0)
    return _tc_layer0(self0, sum0, W_self0, W_neigh0, W_fc, b_fc)


# speedup vs baseline: 1.4872x; 1.0017x over previous
import functools

import jax
import jax.numpy as jnp
from jax import lax
from jax.experimental import pallas as pl
from jax.experimental.pallas import tpu as pltpu
from jax.experimental.pallas import tpu_sc as plsc

NC = 2
NS = 16
NW = NC * NS
L = 16
S = 8
NBUF = 3


def _wid():
    return lax.axis_index("s") * NC + lax.axis_index("c")


def _reduce_by_8(rows_v, out_v, n_nodes, d):
    nblk = d // L

    @plsc.parallel_loop(0, n_nodes, step=1, unroll=2)
    def _(n):
        base = n * S
        for db in range(nblk):
            sl = pl.ds(db * L, L)
            s0 = rows_v[base + 0, sl] + rows_v[base + 1, sl]
            s1 = rows_v[base + 2, sl] + rows_v[base + 3, sl]
            s2 = rows_v[base + 4, sl] + rows_v[base + 5, sl]
            s3 = rows_v[base + 6, sl] + rows_v[base + 7, sl]
            out_v[n, sl] = (s0 + s1) + (s2 + s3)


def _sage_gather_l1(n1p, b_per_w, in_dim, feats, nidx, sidx1):
    n_chunks = b_per_w // L
    half = b_per_w // 2

    mesh = plsc.VectorSubcoreMesh(core_axis_name="c", subcore_axis_name="s")

    @functools.partial(
        pl.kernel,
        out_type=(
            jax.ShapeDtypeStruct((n1p, in_dim), jnp.float32),
            jax.ShapeDtypeStruct((n1p, in_dim), jnp.float32),
        ),
        mesh=mesh,
        scratch_types=[
            pltpu.VMEM((n_chunks * L * S,), jnp.int32),
            pltpu.VMEM((2 * half,), jnp.int32),
            [pltpu.VMEM((L * S, in_dim), jnp.float32)] * NBUF,
            [pltpu.VMEM((L, in_dim), jnp.float32)] * 2,
            [pltpu.VMEM((half, in_dim), jnp.float32)] * 2,
            [pltpu.SemaphoreType.DMA] * NBUF,
            [pltpu.SemaphoreType.DMA] * 2,
            [pltpu.SemaphoreType.DMA] * 2,
        ],
    )
    def k(feats_hbm, nidx_hbm, sidx1_hbm,
          sum1_hbm, self1_hbm,
          nidx_v, sidx_v, rows, outs, srows,
          gsems, osems, ssems):
        w = _wid()
        obase = w * b_per_w

        pltpu.sync_copy(
            nidx_hbm.at[pl.ds(w * n_chunks * L * S, n_chunks * L * S)],
            nidx_v)
        pltpu.sync_copy(sidx1_hbm.at[pl.ds(w * 2 * half, 2 * half)], sidx_v)

        sg = [pltpu.make_async_copy(
            feats_hbm.at[sidx_v.at[pl.ds(i * half, half)]], srows[i],
            ssems[i]) for i in range(2)]
        sg[0].start()
        sg[1].start()

        gathers = [None] * NBUF
        for c in range(min(NBUF, n_chunks)):
            gathers[c] = pltpu.make_async_copy(
                feats_hbm.at[nidx_v.at[pl.ds(c * L * S, L * S)]], rows[c],
                gsems[c])
            gathers[c].start()
        owaits = [None, None]
        for c in range(n_chunks):
            cur = c % NBUF
            gathers[cur].wait()
            nx = c + NBUF
            ocur = c % 2
            if owaits[ocur] is not None:
                owaits[ocur].wait()
            _reduce_by_8(rows[cur], outs[ocur], L, in_dim)
            if nx < n_chunks:
                gathers[cur] = pltpu.make_async_copy(
                    feats_hbm.at[nidx_v.at[pl.ds(nx * L * S, L * S)]],
                    rows[cur], gsems[cur])
                gathers[cur].start()
            owaits[ocur] = pltpu.make_async_copy(
                outs[ocur], sum1_hbm.at[pl.ds(obase + c * L, L), :],
                osems[ocur])
            owaits[ocur].start()
        for ow in owaits:
            if ow is not None:
                ow.wait()

        for i in range(2):
            sg[i].wait()
            pltpu.sync_copy(
                srows[i], self1_hbm.at[pl.ds(obase + i * half, half), :])

    return k(feats, nidx, sidx1)


def _sage_gather_l0(b, hid_dim, in_dim, h, nidx, sidx0, feats):
    n0 = b // NW
    n_chunks = n0 // L

    mesh = plsc.VectorSubcoreMesh(core_axis_name="c", subcore_axis_name="s")

    @functools.partial(
        pl.kernel,
        out_type=(
            jax.ShapeDtypeStruct((b, hid_dim), jnp.float32),
            jax.ShapeDtypeStruct((b, in_dim), jnp.float32),
        ),
        mesh=mesh,
        scratch_types=[
            pltpu.VMEM((n_chunks * L * S,), jnp.int32),
            pltpu.VMEM((n0,), jnp.int32),
            [pltpu.VMEM((L * S, hid_dim), jnp.float32)] * 2,
            [pltpu.VMEM((L, hid_dim), jnp.float32)] * 2,
            pltpu.VMEM((n0, in_dim), jnp.float32),
            [pltpu.SemaphoreType.DMA] * 2,
            [pltpu.SemaphoreType.DMA] * 2,
            pltpu.SemaphoreType.DMA,
        ],
    )
    def k(h_hbm, nidx_hbm, sidx0_hbm, feats_hbm, sum0_hbm, self0_hbm,
          nidx_v, s0idx_v, rows, outs, s0row_v, gsems, osems, ssem):
        w = _wid()
        obase = w * n0

        pltpu.sync_copy(
            nidx_hbm.at[pl.ds(w * n_chunks * L * S, n_chunks * L * S)],
            nidx_v)
        pltpu.sync_copy(sidx0_hbm.at[pl.ds(obase, n0)], s0idx_v)
        s0g = pltpu.make_async_copy(feats_hbm.at[s0idx_v], s0row_v, ssem)
        s0g.start()
        gathers = [None] * n_chunks
        for c in range(n_chunks):
            gathers[c] = pltpu.make_async_copy(
                h_hbm.at[nidx_v.at[pl.ds(c * L * S, L * S)]], rows[c],
                gsems[c])
            gathers[c].start()
        owaits = [None] * n_chunks
        for c in range(n_chunks):
            gathers[c].wait()
            _reduce_by_8(rows[c], outs[c], L, hid_dim)
            owaits[c] = pltpu.make_async_copy(
                outs[c], sum0_hbm.at[pl.ds(obase + c * L, L), :],
                osems[c])
            owaits[c].start()
        s0g.wait()
        pltpu.sync_copy(s0row_v, self0_hbm.at[pl.ds(obase, n0), :])
        for ow in owaits:
            ow.wait()

    return k(h, nidx, sidx0, feats)


def _agg_matmul_kernel(x_ref, m_ref, ws_ref, wn_ref, o_ref):
    o_ref[...] = jnp.maximum(
        jnp.dot(x_ref[...], ws_ref[...], preferred_element_type=jnp.float32)
        + jnp.dot(m_ref[...] * (1.0 / S), wn_ref[...],
                  preferred_element_type=jnp.float32),
        0.0)


def _tc_layer1(x, m, ws, wn, bm=512):
    n1p = x.shape[0]
    hid = ws.shape[1]
    return pl.pallas_call(
        _agg_matmul_kernel,
        grid=(n1p // bm,),
        in_specs=[
            pl.BlockSpec((bm, x.shape[1]), lambda i: (i, 0)),
            pl.BlockSpec((bm, m.shape[1]), lambda i: (i, 0)),
            pl.BlockSpec(ws.shape, lambda i: (0, 0)),
            pl.BlockSpec(wn.shape, lambda i: (0, 0)),
        ],
        out_specs=pl.BlockSpec((bm, hid), lambda i: (i, 0)),
        out_shape=jax.ShapeDtypeStruct((n1p, hid), jnp.float32),
    )(x, m, ws, wn)


def _final_kernel(x_ref, m_ref, ws_ref, wn_ref, wfc_ref, b_ref, o_ref):
    h0 = jnp.maximum(
        jnp.dot(x_ref[...], ws_ref[...], preferred_element_type=jnp.float32)
        + jnp.dot(m_ref[...] * (1.0 / S), wn_ref[...],
                  preferred_element_type=jnp.float32),
        0.0)
    o_ref[...] = (
        jnp.dot(h0, wfc_ref[...], preferred_element_type=jnp.float32)
        + b_ref[...])


def _tc_layer0(x, m, ws, wn, wfc, bfc):
    b = x.shape[0]
    out_dim = wfc.shape[1]
    return pl.pallas_call(
        _final_kernel,
        out_shape=jax.ShapeDtypeStruct((b, out_dim), jnp.float32),
    )(x, m, ws, wn, wfc, bfc.reshape(1, -1))


@jax.jit
def kernel(feats, nodes0, neig0, nodes1, neig1,
           W_self0, W_neigh0, W_self1, W_neigh1, W_fc, b_fc):
    n1 = nodes1.shape[0]
    in_dim = feats.shape[1]
    b = nodes0.shape[0]
    n1p = ((n1 + NW * L - 1) // (NW * L)) * (NW * L)
    pad = n1p - n1
    b_per_w = n1p // NW

    nidx1 = jnp.pad(neig1.astype(jnp.int32), ((0, pad), (0, 0))).reshape(-1)
    sidx1 = jnp.pad(nodes1.astype(jnp.int32), (0, pad))
    sidx0 = nodes0.astype(jnp.int32)
    nidx0 = neig0.astype(jnp.int32).reshape(-1)

    sum1, self1 = _sage_gather_l1(n1p, b_per_w, in_dim, feats, nidx1, sidx1)
    h = _tc_layer1(self1, sum1, W_self1, W_neigh1)
    sum0, self0 = _sage_gather_l0(b, h.shape[1], in_dim, h, nidx0, sidx0,
                                  feats)
    return _tc_layer0(self0, sum0, W_self0, W_neigh0, W_fc, b_fc)


# speedup vs baseline: 2.6038x; 1.7508x over previous
import functools

import jax
import jax.numpy as jnp
from jax import lax
from jax.experimental import pallas as pl
from jax.experimental.pallas import tpu as pltpu
from jax.experimental.pallas import tpu_sc as plsc

NC = 2
NS = 16
NW = NC * NS
L = 16
S = 8
NBUF = 2


def _wid():
    return lax.axis_index("s") * NC + lax.axis_index("c")


def _reduce_by_8(rows_v, out_v, n_nodes, d, out_base=0):
    nblk = d // L

    @plsc.parallel_loop(0, n_nodes, step=1, unroll=2)
    def _(n):
        base = n * S
        for db in range(nblk):
            sl = pl.ds(db * L, L)
            s0 = rows_v[base + 0, sl] + rows_v[base + 1, sl]
            s1 = rows_v[base + 2, sl] + rows_v[base + 3, sl]
            s2 = rows_v[base + 4, sl] + rows_v[base + 5, sl]
            s3 = rows_v[base + 6, sl] + rows_v[base + 7, sl]
            out_v[out_base + n, sl] = (s0 + s1) + (s2 + s3)


def _sage_gather_l1(n1p, b_per_w, in_dim, n_feat, feats, nidx, sidx1):
    nch = 8
    cl = nch * S
    n_chunks = b_per_w // nch
    half = b_per_w // 2
    stage = (n_feat // NS) // 8 * 8
    stage_last = n_feat - stage * (NS - 1)

    mesh = plsc.VectorSubcoreMesh(core_axis_name="c", subcore_axis_name="s")

    @functools.partial(
        pl.kernel,
        out_type=(
            jax.ShapeDtypeStruct((n1p, in_dim), jnp.float32),
            jax.ShapeDtypeStruct((n1p, in_dim), jnp.float32),
        ),
        mesh=mesh,
        scratch_types=[
            pltpu.VMEM_SHARED((n_feat, in_dim), jnp.float32),
            pltpu.VMEM((n_chunks * cl,), jnp.int32),
            pltpu.VMEM((2 * half,), jnp.int32),
            [pltpu.VMEM((cl, in_dim), jnp.float32)] * NBUF,
            pltpu.VMEM((b_per_w, in_dim), jnp.float32),
            [pltpu.SemaphoreType.DMA] * NBUF,
            [pltpu.SemaphoreType.DMA] * 2,
        ],
    )
    def k(feats_hbm, nidx_hbm, sidx1_hbm,
          sum1_hbm, self1_hbm,
          table, nidx_v, sidx_v, rows, outbuf,
          gsems, osems):
        w = _wid()
        sid = lax.axis_index("s")
        obase = w * b_per_w

        pltpu.sync_copy(
            feats_hbm.at[pl.ds(sid * stage, stage), :],
            table.at[pl.ds(sid * stage, stage), :])
        rem = stage_last - stage
        if rem > 0:
            @pl.when(sid == NS - 1)
            def _():
                pltpu.sync_copy(
                    feats_hbm.at[pl.ds(NS * stage, rem), :],
                    table.at[pl.ds(NS * stage, rem), :])

        pltpu.sync_copy(
            nidx_hbm.at[pl.ds(w * n_chunks * cl, n_chunks * cl)],
            nidx_v)
        pltpu.sync_copy(sidx1_hbm.at[pl.ds(w * 2 * half, 2 * half)], sidx_v)

        plsc.subcore_barrier()

        def start_gather(buf, chunk_start):
            off = chunk_start * cl
            if not isinstance(off, int):
                off = pl.multiple_of(off, 8)
            g = pltpu.make_async_copy(
                table.at[nidx_v.at[pl.ds(off, cl)]], rows[buf],
                gsems[buf])
            g.start()
            return g

        start_gather(0, 0)
        start_gather(1, 1)

        def pair_body(j, carry):
            c0 = j * 2
            for buf in range(2):
                c = c0 + buf
                pltpu.make_async_copy(
                    table.at[nidx_v.at[pl.ds(0, cl)]], rows[buf],
                    gsems[buf]).wait()
                _reduce_by_8(rows[buf], outbuf, nch, in_dim,
                             out_base=c * nch)

                @pl.when(c + 2 < n_chunks)
                def _():
                    start_gather(buf, c + 2)
            return carry

        lax.fori_loop(0, n_chunks // 2, pair_body, 0)

        pltpu.sync_copy(outbuf, sum1_hbm.at[pl.ds(obase, b_per_w), :])

        segs = []
        off = 0
        while off < b_per_w:
            seg = min(cl, b_per_w - off)
            segs.append((off, seg))
            off += seg
        sgs = [None] * NBUF
        for i, (off, seg) in enumerate(segs):
            buf = i % NBUF
            if sgs[buf] is not None:
                sgs[buf][0].wait()
                pltpu.sync_copy(
                    rows[buf].at[pl.ds(0, sgs[buf][2]), :],
                    self1_hbm.at[pl.ds(obase + sgs[buf][1],
                                       sgs[buf][2]), :])
            g = pltpu.make_async_copy(
                table.at[sidx_v.at[pl.ds(off, seg)]],
                rows[buf].at[pl.ds(0, seg), :], gsems[buf])
            g.start()
            sgs[buf] = (g, off, seg)
        for buf in range(NBUF):
            if sgs[buf] is not None:
                sgs[buf][0].wait()
                pltpu.sync_copy(
                    rows[buf].at[pl.ds(0, sgs[buf][2]), :],
                    self1_hbm.at[pl.ds(obase + sgs[buf][1],
                                       sgs[buf][2]), :])

    return k(feats, nidx, sidx1)


def _sage_gather_l0(b, hid_dim, in_dim, h, nidx, sidx0, feats):
    n0 = b // NW
    n_chunks = n0 // L

    mesh = plsc.VectorSubcoreMesh(core_axis_name="c", subcore_axis_name="s")

    @functools.partial(
        pl.kernel,
        out_type=(
            jax.ShapeDtypeStruct((b, hid_dim), jnp.float32),
            jax.ShapeDtypeStruct((b, in_dim), jnp.float32),
        ),
        mesh=mesh,
        scratch_types=[
            pltpu.VMEM((n_chunks * L * S,), jnp.int32),
            pltpu.VMEM((n0,), jnp.int32),
            [pltpu.VMEM((L * S, hid_dim), jnp.float32)] * 2,
            [pltpu.VMEM((L, hid_dim), jnp.float32)] * 2,
            pltpu.VMEM((n0, in_dim), jnp.float32),
            [pltpu.SemaphoreType.DMA] * 2,
            [pltpu.SemaphoreType.DMA] * 2,
            pltpu.SemaphoreType.DMA,
        ],
    )
    def k(h_hbm, nidx_hbm, sidx0_hbm, feats_hbm, sum0_hbm, self0_hbm,
          nidx_v, s0idx_v, rows, outs, s0row_v, gsems, osems, ssem):
        w = _wid()
        obase = w * n0

        pltpu.sync_copy(
            nidx_hbm.at[pl.ds(w * n_chunks * L * S, n_chunks * L * S)],
            nidx_v)
        pltpu.sync_copy(sidx0_hbm.at[pl.ds(obase, n0)], s0idx_v)
        s0g = pltpu.make_async_copy(feats_hbm.at[s0idx_v], s0row_v, ssem)
        s0g.start()
        gathers = [None] * n_chunks
        for c in range(n_chunks):
            gathers[c] = pltpu.make_async_copy(
                h_hbm.at[nidx_v.at[pl.ds(c * L * S, L * S)]], rows[c],
                gsems[c])
            gathers[c].start()
        owaits = [None] * n_chunks
        for c in range(n_chunks):
            gathers[c].wait()
            _reduce_by_8(rows[c], outs[c], L, hid_dim)
            owaits[c] = pltpu.make_async_copy(
                outs[c], sum0_hbm.at[pl.ds(obase + c * L, L), :],
                osems[c])
            owaits[c].start()
        s0g.wait()
        pltpu.sync_copy(s0row_v, self0_hbm.at[pl.ds(obase, n0), :])
        for ow in owaits:
            ow.wait()

    return k(h, nidx, sidx0, feats)


def _agg_matmul_kernel(x_ref, m_ref, ws_ref, wn_ref, o_ref):
    o_ref[...] = jnp.maximum(
        jnp.dot(x_ref[...], ws_ref[...], preferred_element_type=jnp.float32)
        + jnp.dot(m_ref[...] * (1.0 / S), wn_ref[...],
                  preferred_element_type=jnp.float32),
        0.0)


def _tc_layer1(x, m, ws, wn, bm=512):
    n1p = x.shape[0]
    hid = ws.shape[1]
    return pl.pallas_call(
        _agg_matmul_kernel,
        grid=(n1p // bm,),
        in_specs=[
            pl.BlockSpec((bm, x.shape[1]), lambda i: (i, 0)),
            pl.BlockSpec((bm, m.shape[1]), lambda i: (i, 0)),
            pl.BlockSpec(ws.shape, lambda i: (0, 0)),
            pl.BlockSpec(wn.shape, lambda i: (0, 0)),
        ],
        out_specs=pl.BlockSpec((bm, hid), lambda i: (i, 0)),
        out_shape=jax.ShapeDtypeStruct((n1p, hid), jnp.float32),
    )(x, m, ws, wn)


def _final_kernel(x_ref, m_ref, ws_ref, wn_ref, wfc_ref, b_ref, o_ref):
    h0 = jnp.maximum(
        jnp.dot(x_ref[...], ws_ref[...], preferred_element_type=jnp.float32)
        + jnp.dot(m_ref[...] * (1.0 / S), wn_ref[...],
                  preferred_element_type=jnp.float32),
        0.0)
    o_ref[...] = (
        jnp.dot(h0, wfc_ref[...], preferred_element_type=jnp.float32)
        + b_ref[...])


def _tc_layer0(x, m, ws, wn, wfc, bfc):
    b = x.shape[0]
    out_dim = wfc.shape[1]
    return pl.pallas_call(
        _final_kernel,
        out_shape=jax.ShapeDtypeStruct((b, out_dim), jnp.float32),
    )(x, m, ws, wn, wfc, bfc.reshape(1, -1))


@jax.jit
def kernel(feats, nodes0, neig0, nodes1, neig1,
           W_self0, W_neigh0, W_self1, W_neigh1, W_fc, b_fc):
    n1 = nodes1.shape[0]
    in_dim = feats.shape[1]
    b = nodes0.shape[0]
    n1p = ((n1 + NW * L - 1) // (NW * L)) * (NW * L)
    pad = n1p - n1
    b_per_w = n1p // NW

    nidx1 = jnp.pad(neig1.astype(jnp.int32), ((0, pad), (0, 0))).reshape(-1)
    sidx1 = jnp.pad(nodes1.astype(jnp.int32), (0, pad))
    sidx0 = nodes0.astype(jnp.int32)
    nidx0 = neig0.astype(jnp.int32).reshape(-1)

    sum1, self1 = _sage_gather_l1(n1p, b_per_w, in_dim, feats.shape[0],
                                  feats, nidx1, sidx1)
    h = _tc_layer1(self1, sum1, W_self1, W_neigh1)
    sum0, self0 = _sage_gather_l0(b, h.shape[1], in_dim, h, nidx0, sidx0,
                                  feats)
    return _tc_layer0(self0, sum0, W_self0, W_neigh0, W_fc, b_fc)
